# Initial kernel scaffold; baseline (speedup 1.0000x reference)
import functools

import jax
import jax.numpy as jnp
from jax import lax
from jax.experimental import pallas as pl
from jax.experimental.pallas import tpu as pltpu
from jax.experimental.pallas import tpu_sc as plsc

N_NODES_K = 10000
N_EDGES_K = 160000
D_FEAT_K = 256

CHUNK = 128
ROWS2D = 1280
PAD_E = ROWS2D * CHUNK - N_EDGES_K
HALF = N_NODES_K // 2
ZROWS = 5120
ZTRASH = HALF
DEG_ROWS = 10240
DEG_TRASH = N_NODES_K
BLK = 200

_MESH = plsc.VectorSubcoreMesh(core_axis_name="c", subcore_axis_name="s")


@functools.partial(
    pl.kernel,
    out_type=jax.ShapeDtypeStruct((2, DEG_ROWS, 16), jnp.float32),
    mesh=_MESH,
    scratch_types=[
        pltpu.VMEM_SHARED((DEG_ROWS, 16), jnp.float32),
        pltpu.VMEM((40, CHUNK), jnp.int32),
        pltpu.VMEM((40, CHUNK), jnp.int32),
        pltpu.VMEM((CHUNK, 16), jnp.float32),
        pltpu.VMEM((640, 16), jnp.float32),
    ],
)
def _sc_degree(row_hbm, col_hbm, out_hbm, deg_sh, rbuf, cbuf, ones, zbuf):
    cid = lax.axis_index("c")
    sid = lax.axis_index("s")

    @pl.loop(0, CHUNK)
    def _(i):
        ones[i, :] = jnp.full((16,), 1.0, jnp.float32)

    @pl.loop(0, 640)
    def _(i):
        zbuf[i, :] = jnp.zeros((16,), jnp.float32)

    pltpu.sync_copy(zbuf, deg_sh.at[pl.ds(sid * 640, 640)])
    plsc.subcore_barrier()

    base_r = cid * 640 + sid * 40
    pltpu.sync_copy(row_hbm.at[pl.ds(base_r, 40)], rbuf)
    pltpu.sync_copy(col_hbm.at[pl.ds(base_r, 40)], cbuf)

    @pl.loop(0, 40)
    def _(i):
        @pl.loop(0, CHUNK, step=16)
        def _(j):
            c16 = cbuf[i, pl.ds(j, 16)]
            r16 = rbuf[i, pl.ds(j, 16)]
            cbuf[i, pl.ds(j, 16)] = jnp.where(
                c16 == r16, jnp.full((16,), DEG_TRASH, jnp.int32), c16
            )

    @pl.loop(0, 40)
    def _(k):
        pltpu.sync_copy(ones, deg_sh.at[cbuf.at[k]], add=True)

    plsc.subcore_barrier()
    pltpu.sync_copy(
        deg_sh.at[pl.ds(sid * 640, 640)],
        out_hbm.at[cid, pl.ds(sid * 640, 640)],
    )


@functools.partial(
    pl.kernel,
    out_type=jax.ShapeDtypeStruct((2, ZROWS, D_FEAT_K), jnp.float32),
    mesh=_MESH,
    scratch_types=[
        pltpu.VMEM_SHARED((ZROWS, D_FEAT_K), jnp.float32),
        pltpu.VMEM((80, CHUNK), jnp.int32),
        pltpu.VMEM((80, CHUNK), jnp.int32),
        pltpu.VMEM((CHUNK, D_FEAT_K), jnp.float32),
        pltpu.VMEM((CHUNK, D_FEAT_K), jnp.float32),
        pltpu.SemaphoreType.DMA,
        pltpu.SemaphoreType.DMA,
    ],
)
def _sc_scatter(row_hbm, col_hbm, y_hbm, out_hbm, z_sh, rbuf, cbuf, g0, g1,
                sem0, sem1):
    cid = lax.axis_index("c")
    sid = lax.axis_index("s")
    nk = 80

    base_r = sid * nk
    pltpu.sync_copy(row_hbm.at[pl.ds(base_r, nk)], rbuf)
    pltpu.sync_copy(col_hbm.at[pl.ds(base_r, nk)], cbuf)

    cbase = cid * HALF

    @pl.loop(0, nk)
    def _(i):
        @pl.loop(0, CHUNK, step=16)
        def _(j):
            c16 = cbuf[i, pl.ds(j, 16)]
            r16 = rbuf[i, pl.ds(j, 16)]
            cl = c16 - cbase
            bad = (cl < 0) | (cl >= HALF) | (c16 == r16)
            cbuf[i, pl.ds(j, 16)] = jnp.where(
                bad, jnp.full((16,), ZTRASH, jnp.int32), cl
            )

    @pl.loop(0, CHUNK)
    def _(i):
        @pl.loop(0, D_FEAT_K, step=16)
        def _(j):
            g0[i, pl.ds(j, 16)] = jnp.zeros((16,), jnp.float32)

    zb = sid * 320
    pltpu.sync_copy(g0, z_sh.at[pl.ds(zb, 128)])
    pltpu.sync_copy(g0, z_sh.at[pl.ds(zb + 128, 128)])
    pltpu.sync_copy(g0.at[pl.ds(0, 64)], z_sh.at[pl.ds(zb + 256, 64)])
    plsc.subcore_barrier()

    def _start(k, gbuf, sem):
        pltpu.make_async_copy(y_hbm.at[rbuf.at[k]], gbuf, sem).start()

    def _finish(k, gbuf, sem):
        pltpu.make_async_copy(y_hbm.at[rbuf.at[k]], gbuf, sem).wait()
        pltpu.sync_copy(gbuf, z_sh.at[cbuf.at[k]], add=True)

    _start(0, g0, sem0)
    _start(1, g1, sem1)

    @pl.loop(0, nk, step=2)
    def _(k):
        _finish(k, g0, sem0)

        @pl.when(k + 2 < nk)
        def _():
            _start(k + 2, g0, sem0)

        _finish(k + 1, g1, sem1)

        @pl.when(k + 3 < nk)
        def _():
            _start(k + 3, g1, sem1)

    plsc.subcore_barrier()
    pltpu.sync_copy(
        z_sh.at[pl.ds(zb, 320)], out_hbm.at[cid, pl.ds(zb, 320)]
    )


def _t1_body(d0, d1, x, y):
    deg = d0[...][:, 0:1] + d1[...][:, 0:1] + 1.0
    dinv = lax.rsqrt(deg)
    y[...] = dinv * x[...]


def _t2_body(d0, d1, z, yy, o):
    deg = d0[...][:, 0:1] + d1[...][:, 0:1] + 1.0
    dinv = lax.rsqrt(deg)
    o[...] = dinv * (z[...][0] + yy[...])


def kernel(edge_index, input_feature):
    ei = edge_index.astype(jnp.int32)
    row, col = ei[0], ei[1]
    rowp = jnp.concatenate(
        [row, jnp.zeros((PAD_E,), jnp.int32)]).reshape(ROWS2D, CHUNK)
    colp = jnp.concatenate(
        [col, jnp.full((PAD_E,), N_NODES_K, jnp.int32)]).reshape(ROWS2D, CHUNK)

    degp = _sc_degree(rowp, colp)
    d0, d1 = degp[0], degp[1]

    nblk = N_NODES_K // BLK
    y = pl.pallas_call(
        _t1_body,
        out_shape=jax.ShapeDtypeStruct((N_NODES_K, D_FEAT_K), jnp.float32),
        grid=(nblk,),
        in_specs=[
            pl.BlockSpec((BLK, 16), lambda i: (i, 0)),
            pl.BlockSpec((BLK, 16), lambda i: (i, 0)),
            pl.BlockSpec((BLK, D_FEAT_K), lambda i: (i, 0)),
        ],
        out_specs=pl.BlockSpec((BLK, D_FEAT_K), lambda i: (i, 0)),
    )(d0, d1, input_feature)

    zz = _sc_scatter(rowp, colp, y)

    out = pl.pallas_call(
        _t2_body,
        out_shape=jax.ShapeDtypeStruct((N_NODES_K, D_FEAT_K), jnp.float32),
        grid=(nblk,),
        in_specs=[
            pl.BlockSpec((BLK, 16), lambda i: (i, 0)),
            pl.BlockSpec((BLK, 16), lambda i: (i, 0)),
            pl.BlockSpec(
                (1, BLK, D_FEAT_K),
                lambda i: (i // (HALF // BLK), i % (HALF // BLK), 0),
            ),
            pl.BlockSpec((BLK, D_FEAT_K), lambda i: (i, 0)),
        ],
        out_specs=pl.BlockSpec((BLK, D_FEAT_K), lambda i: (i, 0)),
    )(d0, d1, zz, y)
    return out


# speedup vs baseline: 8.7613x; 8.7613x over previous
import functools

import jax
import jax.numpy as jnp
from jax import lax
from jax.experimental import pallas as pl
from jax.experimental.pallas import tpu as pltpu
from jax.experimental.pallas import tpu_sc as plsc

N_NODES_K = 10000
N_EDGES_K = 160000
D_FEAT_K = 256
HALF_D = 128

CHUNK = 64
ROWS2D = 2560
PAD_E = ROWS2D * CHUNK - N_EDGES_K
NKS = ROWS2D // 16
ZROWS = 10240
ZTRASH = N_NODES_K
DEG_ROWS = 10240
DEG_TRASH = N_NODES_K
BLK = 200

_MESH = plsc.VectorSubcoreMesh(core_axis_name="c", subcore_axis_name="s")


@functools.partial(
    pl.kernel,
    out_type=jax.ShapeDtypeStruct((2, DEG_ROWS, HALF_D), jnp.float32),
    mesh=_MESH,
    scratch_types=[
        pltpu.VMEM_SHARED((DEG_ROWS, HALF_D), jnp.float32),
        pltpu.VMEM((80, CHUNK), jnp.int32),
        pltpu.VMEM((80, CHUNK), jnp.int32),
        pltpu.VMEM((CHUNK,), jnp.int32),
        pltpu.VMEM((CHUNK, HALF_D), jnp.float32),
        pltpu.VMEM((CHUNK, HALF_D), jnp.float32),
    ],
)
def _sc_degree(row_hbm, col_hbm, out_hbm, deg_sh, rbuf, cbuf, cslot, ones,
               zbuf):
    cid = lax.axis_index("c")
    sid = lax.axis_index("s")

    @pl.loop(0, CHUNK)
    def _(i):
        @pl.loop(0, HALF_D, step=16)
        def _(j):
            ones[i, pl.ds(j, 16)] = jnp.full((16,), 1.0, jnp.float32)
            zbuf[i, pl.ds(j, 16)] = jnp.zeros((16,), jnp.float32)

    for part in range(10):
        pltpu.sync_copy(
            zbuf, deg_sh.at[pl.ds(sid * 640 + part * CHUNK, CHUNK)])
    plsc.subcore_barrier()

    base_r = cid * 1280 + sid * 80
    pltpu.sync_copy(row_hbm.at[pl.ds(base_r, 80)], rbuf)
    pltpu.sync_copy(col_hbm.at[pl.ds(base_r, 80)], cbuf)

    trash16 = DEG_TRASH + lax.iota(jnp.int32, 16)

    @pl.loop(0, 80)
    def _(k):
        @pl.loop(0, CHUNK, step=16)
        def _(j):
            c16 = cbuf[k, pl.ds(j, 16)]
            r16 = rbuf[k, pl.ds(j, 16)]
            cslot[pl.ds(j, 16)] = jnp.where(c16 == r16, trash16, c16)

        pltpu.sync_copy(ones, deg_sh.at[cslot], add=True)

    plsc.subcore_barrier()
    pltpu.sync_copy(
        deg_sh.at[pl.ds(sid * 640, 640)],
        out_hbm.at[cid, pl.ds(sid * 640, 640)],
    )


@functools.partial(
    pl.kernel,
    out_type=jax.ShapeDtypeStruct((2, ZROWS, HALF_D), jnp.float32),
    mesh=_MESH,
    scratch_types=[
        pltpu.VMEM_SHARED((ZROWS, HALF_D), jnp.float32),
        pltpu.VMEM((2, CHUNK), jnp.int32),
        pltpu.VMEM((2, CHUNK), jnp.int32),
        pltpu.VMEM((CHUNK,), jnp.int32),
        pltpu.VMEM((CHUNK,), jnp.int32),
        pltpu.VMEM((CHUNK,), jnp.int32),
        pltpu.VMEM((CHUNK,), jnp.int32),
        pltpu.VMEM((CHUNK, HALF_D), jnp.float32),
        pltpu.VMEM((CHUNK, HALF_D), jnp.float32),
        pltpu.SemaphoreType.DMA,
        pltpu.SemaphoreType.DMA,
        pltpu.SemaphoreType.DMA,
        pltpu.SemaphoreType.DMA,
    ],
)
def _sc_scatter(e3_hbm, y_hbm, out_hbm, z_sh, i0, i1, r0, r1, c0, c1, g0, g1,
                is0, is1, gs0, gs1):
    cid = lax.axis_index("c")
    sid = lax.axis_index("s")
    ybase = cid * N_NODES_K
    trash16 = ZTRASH + lax.iota(jnp.int32, 16)

    def _fix(islot, rslot, cslot):
        @pl.loop(0, CHUNK, step=16)
        def _(j):
            c16 = islot[1, pl.ds(j, 16)]
            r16 = islot[0, pl.ds(j, 16)]
            rslot[pl.ds(j, 16)] = r16 + ybase
            bad = (c16 >= N_NODES_K) | (c16 == r16)
            cslot[pl.ds(j, 16)] = jnp.where(bad, trash16, c16)

    def _idx_start(k, islot, isem):
        pltpu.make_async_copy(e3_hbm.at[sid * NKS + k], islot, isem).start()

    def _idx_wait(k, islot, isem):
        pltpu.make_async_copy(e3_hbm.at[sid * NKS + k], islot, isem).wait()

    def _g_start(rslot, gbuf, gsem):
        pltpu.make_async_copy(y_hbm.at[rslot], gbuf, gsem).start()

    def _g_wait(rslot, gbuf, gsem):
        pltpu.make_async_copy(y_hbm.at[rslot], gbuf, gsem).wait()

    @pl.loop(0, CHUNK)
    def _(i):
        @pl.loop(0, HALF_D, step=16)
        def _(j):
            g0[i, pl.ds(j, 16)] = jnp.zeros((16,), jnp.float32)

    zb = sid * 640
    for part in range(10):
        pltpu.sync_copy(g0, z_sh.at[pl.ds(zb + part * CHUNK, CHUNK)])
    plsc.subcore_barrier()

    pltpu.sync_copy(e3_hbm.at[sid * NKS], i0)
    _fix(i0, r0, c0)
    _g_start(r0, g0, gs0)
    _idx_start(1, i1, is1)

    @pl.loop(0, NKS, step=2)
    def _(k):
        _idx_wait(k + 1, i1, is1)
        _fix(i1, r1, c1)

        @pl.when(k + 2 < NKS)
        def _():
            _idx_start(k + 2, i0, is0)

        _g_start(r1, g1, gs1)
        _g_wait(r0, g0, gs0)
        pltpu.sync_copy(g0, z_sh.at[c0], add=True)

        @pl.when(k + 2 < NKS)
        def _():
            _idx_wait(k + 2, i0, is0)
            _fix(i0, r0, c0)

            @pl.when(k + 3 < NKS)
            def _():
                _idx_start(k + 3, i1, is1)

            _g_start(r0, g0, gs0)

        _g_wait(r1, g1, gs1)
        pltpu.sync_copy(g1, z_sh.at[c1], add=True)

    plsc.subcore_barrier()
    pltpu.sync_copy(
        z_sh.at[pl.ds(zb, 640)], out_hbm.at[cid, pl.ds(zb, 640)]
    )


def _t1_body(d0, d1, x, y):
    deg = d0[...][:, 0:1] + d1[...][:, 0:1] + 1.0
    dinv = lax.rsqrt(deg)
    y[...] = dinv * x[...]


def _t2_write(d0, d1, z, yy, o):
    deg = d0[...][:, 0:1] + d1[...][:, 0:1] + 1.0
    dinv = lax.rsqrt(deg)
    lo = dinv * (z[...][0] + yy[...][0])
    hi = dinv * (z[...][1] + yy[...][1])
    o[...] = jnp.concatenate([lo, hi], axis=1)


def kernel(edge_index, input_feature):
    ei = edge_index.astype(jnp.int32)
    row, col = ei[0], ei[1]
    rowp = jnp.concatenate(
        [row, jnp.zeros((PAD_E,), jnp.int32)]).reshape(ROWS2D, CHUNK)
    colp = jnp.concatenate(
        [col, jnp.full((PAD_E,), N_NODES_K, jnp.int32)]).reshape(ROWS2D, CHUNK)
    e3 = jnp.stack([rowp, colp], axis=1)

    degp = _sc_degree(rowp, colp)
    d0 = degp[0, :N_NODES_K, :16]
    d1 = degp[1, :N_NODES_K, :16]

    nblk = N_NODES_K // BLK
    y = pl.pallas_call(
        _t1_body,
        out_shape=jax.ShapeDtypeStruct((2 * N_NODES_K, HALF_D), jnp.float32),
        grid=(2, nblk),
        in_specs=[
            pl.BlockSpec((BLK, 16), lambda h, i: (i, 0)),
            pl.BlockSpec((BLK, 16), lambda h, i: (i, 0)),
            pl.BlockSpec((BLK, HALF_D), lambda h, i: (i, h)),
        ],
        out_specs=pl.BlockSpec(
            (BLK, HALF_D), lambda h, i: (h * (N_NODES_K // BLK) + i, 0)),
    )(d0, d1, input_feature)

    zz = _sc_scatter(e3, y)
    z2 = zz[:, :N_NODES_K]
    y2 = y.reshape(2, N_NODES_K, HALF_D)

    out = pl.pallas_call(
        _t2_write,
        out_shape=jax.ShapeDtypeStruct((N_NODES_K, D_FEAT_K), jnp.float32),
        grid=(nblk,),
        in_specs=[
            pl.BlockSpec((BLK, 16), lambda i: (i, 0)),
            pl.BlockSpec((BLK, 16), lambda i: (i, 0)),
            pl.BlockSpec((2, BLK, HALF_D), lambda i: (0, i, 0)),
            pl.BlockSpec((2, BLK, HALF_D), lambda i: (0, i, 0)),
        ],
        out_specs=pl.BlockSpec((BLK, D_FEAT_K), lambda i: (i, 0)),
    )(d0, d1, z2, y2)
    return out
